# baseline (speedup 1.0000x reference)
import functools

import jax
import jax.numpy as jnp
from jax import lax
from jax.experimental import pallas as pl
from jax.experimental.pallas import tpu as pltpu
from jax.experimental.pallas import tpu_sc as plsc

NUM_CLASSES = 1000000
EMB_CHANNEL = 64
BATCH = 16384
HALF = NUM_CLASSES // 2

NC = 2
NS = 16
NW = NC * NS
B_PER_W = BATCH // NW
CHUNK = 128
NCHUNK = B_PER_W // CHUNK
L = 16


@functools.lru_cache(maxsize=1)
def _build_gather():
    mesh = plsc.VectorSubcoreMesh(core_axis_name="c", subcore_axis_name="s")

    @functools.partial(
        pl.kernel,
        mesh=mesh,
        out_type=jax.ShapeDtypeStruct((BATCH, EMB_CHANNEL), jnp.float32),
        scratch_types=[
            pltpu.VMEM((NCHUNK, CHUNK), jnp.int32),
            pltpu.VMEM((NCHUNK, CHUNK), jnp.int32),
            pltpu.VMEM((B_PER_W,), jnp.int32),
            pltpu.VMEM((B_PER_W, EMB_CHANNEL), jnp.float32),
            pltpu.VMEM((B_PER_W, EMB_CHANNEL), jnp.float32),
            pltpu.SemaphoreType.DMA,
            pltpu.SemaphoreType.DMA,
        ],
        compiler_params=pltpu.CompilerParams(
            use_tc_tiling_on_sc=False, needs_layout_passes=False
        ),
    )
    def emb_gather(half0_hbm, half1_hbm, idx0_hbm, idx1_hbm, mask_hbm, out_hbm,
                   idx0_v, idx1_v, mask_v, rows0_v, rows1_v, sem0, sem1):
        wid = lax.axis_index("s") * NC + lax.axis_index("c")
        base_c = wid * NCHUNK
        base = wid * B_PER_W
        pltpu.sync_copy(idx0_hbm.at[pl.ds(base_c, NCHUNK)], idx0_v)
        pltpu.sync_copy(idx1_hbm.at[pl.ds(base_c, NCHUNK)], idx1_v)
        pltpu.sync_copy(mask_hbm.at[pl.ds(base, B_PER_W)], mask_v)
        for j in range(NCHUNK):
            pltpu.async_copy(
                half0_hbm.at[idx0_v.at[j]],
                rows0_v.at[pl.ds(j * CHUNK, CHUNK)],
                sem0,
            )
            pltpu.async_copy(
                half1_hbm.at[idx1_v.at[j]],
                rows1_v.at[pl.ds(j * CHUNK, CHUNK)],
                sem1,
            )
        pltpu.make_async_copy(out_hbm.at[pl.ds(base, B_PER_W)], rows0_v, sem0).wait()
        pltpu.make_async_copy(out_hbm.at[pl.ds(base, B_PER_W)], rows1_v, sem1).wait()
        iota = lax.iota(jnp.int32, L)

        def merge_body(g, _):
            row_vec = iota + g * L
            m_vec = mask_v[pl.ds(g * L, L)] != 0
            for k in range(EMB_CHANNEL):
                col = jnp.full((L,), k, jnp.int32)
                v1 = plsc.load_gather(rows1_v, [row_vec, col])
                plsc.store_scatter(rows0_v, [row_vec, col], v1, mask=m_vec)
            return _

        lax.fori_loop(0, B_PER_W // L, merge_body, None)
        pltpu.sync_copy(rows0_v, out_hbm.at[pl.ds(base, B_PER_W)])

    return emb_gather


def kernel(class_labels, label_emb_weight):
    i32 = class_labels.astype(jnp.int32)
    in_hi = i32 >= HALF
    idx0 = jnp.where(in_hi, 0, i32).reshape(NW * NCHUNK, CHUNK)
    idx1 = jnp.where(in_hi, i32 - HALF, 0).reshape(NW * NCHUNK, CHUNK)
    mask = in_hi.astype(jnp.int32)
    half0 = label_emb_weight[:HALF]
    half1 = label_emb_weight[HALF:]
    return _build_gather()(half0, half1, idx0, idx1, mask)


# speedup vs baseline: 1.7194x; 1.7194x over previous
import functools

import jax
import jax.numpy as jnp
from jax import lax
from jax.experimental import pallas as pl
from jax.experimental.pallas import tpu as pltpu
from jax.experimental.pallas import tpu_sc as plsc

NUM_CLASSES = 1000000
EMB_CHANNEL = 64
BATCH = 16384

NC = 2
NS = 16
NW = NC * NS
B_PER_W = BATCH // NW
CHUNK = 128
NCHUNK = B_PER_W // CHUNK
L = 16
AG = EMB_CHANNEL // 8
NBLK = BATCH // CHUNK


@functools.lru_cache(maxsize=1)
def _build_gather():
    mesh = plsc.VectorSubcoreMesh(core_axis_name="c", subcore_axis_name="s")

    @functools.partial(
        pl.kernel,
        mesh=mesh,
        out_type=jax.ShapeDtypeStruct((AG, NBLK, 8, CHUNK), jnp.float32),
        scratch_types=[
            pltpu.VMEM((NCHUNK, CHUNK), jnp.int32),
            pltpu.VMEM((B_PER_W, EMB_CHANNEL), jnp.float32),
            pltpu.VMEM((NCHUNK, 8, CHUNK), jnp.float32),
            pltpu.VMEM((NCHUNK, 8, CHUNK), jnp.float32),
            pltpu.SemaphoreType.DMA,
            pltpu.SemaphoreType.DMA,
        ],
        compiler_params=pltpu.CompilerParams(
            use_tc_tiling_on_sc=False, needs_layout_passes=False
        ),
    )
    def emb_gather(table_hbm, idx_hbm, out_hbm, idx_v, rows_v, xb0, xb1,
                   sem, sem_w):
        wid = lax.axis_index("s") * NC + lax.axis_index("c")
        base_k = wid * NCHUNK
        pltpu.sync_copy(idx_hbm.at[pl.ds(base_k, NCHUNK)], idx_v)
        copies = []
        for j in range(NCHUNK):
            copies.append(
                pltpu.async_copy(
                    table_hbm.at[idx_v.at[j]],
                    rows_v.at[pl.ds(j * CHUNK, CHUNK)],
                    sem,
                )
            )
        for c in copies:
            c.wait()
        iota = lax.iota(jnp.int32, L)
        xbufs = (xb0, xb1)
        for a in range(AG):
            xb = xbufs[a % 2]
            if a >= 2:
                pltpu.make_async_copy(
                    out_hbm.at[a - 2, pl.ds(base_k, NCHUNK)], xb, sem_w
                ).wait()

            def fbody(f, _, a=a, xb=xb):
                col = jnp.full((L,), 0, jnp.int32) + (a * 8 + f)
                for k in range(NCHUNK):
                    for q in range(CHUNK // L):
                        row_vec = iota + (k * CHUNK + q * L)
                        v = plsc.load_gather(rows_v, [row_vec, col])
                        xb[k, f, pl.ds(q * L, L)] = v
                return _

            lax.fori_loop(0, 8, fbody, None)
            pltpu.async_copy(xb, out_hbm.at[a, pl.ds(base_k, NCHUNK)], sem_w)
        pltpu.make_async_copy(
            out_hbm.at[AG - 2, pl.ds(base_k, NCHUNK)], xb0, sem_w
        ).wait()
        pltpu.make_async_copy(
            out_hbm.at[AG - 1, pl.ds(base_k, NCHUNK)], xb1, sem_w
        ).wait()

    return emb_gather


def kernel(class_labels, label_emb_weight):
    idx = class_labels.astype(jnp.int32).reshape(NW * NCHUNK, CHUNK)
    x = _build_gather()(label_emb_weight, idx)
    return x.transpose(1, 3, 0, 2).reshape(BATCH, EMB_CHANNEL)


# speedup vs baseline: 3.0252x; 1.7595x over previous
import functools

import jax
import jax.numpy as jnp
from jax import lax
from jax.experimental import pallas as pl
from jax.experimental.pallas import tpu as pltpu
from jax.experimental.pallas import tpu_sc as plsc

NUM_CLASSES = 1000000
EMB_CHANNEL = 64
BATCH = 16384

NC = 2
NS = 16
NW = NC * NS
B_PER_W = BATCH // NW


@functools.lru_cache(maxsize=1)
def _build_gather():
    mesh = plsc.VectorSubcoreMesh(core_axis_name="c", subcore_axis_name="s")

    @functools.partial(
        pl.kernel,
        mesh=mesh,
        out_type=jax.ShapeDtypeStruct((BATCH, EMB_CHANNEL), jnp.float32),
        scratch_types=[
            pltpu.VMEM_SHARED((NS, B_PER_W), jnp.int32),
            pltpu.SMEM((B_PER_W,), jnp.int32),
            pltpu.VMEM((B_PER_W, EMB_CHANNEL), jnp.float32),
            pltpu.SemaphoreType.DMA,
        ],
    )
    def emb_gather(table_hbm, idx_hbm, out_hbm, idx_sh, idx_s, rows_v, sem):
        cid = lax.axis_index("c")
        sid = lax.axis_index("s")
        wid = sid * NC + cid
        base = wid * B_PER_W
        pltpu.sync_copy(idx_hbm.at[pl.ds(base, B_PER_W)], idx_sh.at[sid])
        pltpu.sync_copy(idx_sh.at[sid], idx_s)

        def body(j, _):
            pltpu.async_copy(table_hbm.at[idx_s[j]], rows_v.at[j], sem)
            return _

        lax.fori_loop(0, B_PER_W, body, None)
        pltpu.make_async_copy(
            out_hbm.at[pl.ds(base, B_PER_W)], rows_v, sem
        ).wait()
        pltpu.sync_copy(rows_v, out_hbm.at[pl.ds(base, B_PER_W)])

    return emb_gather


def kernel(class_labels, label_emb_weight):
    idx = class_labels.astype(jnp.int32)
    return _build_gather()(label_emb_weight, idx)


# speedup vs baseline: 4.4759x; 1.4795x over previous
import functools

import jax
import jax.numpy as jnp
from jax import lax
from jax.experimental import pallas as pl
from jax.experimental.pallas import tpu as pltpu
from jax.experimental.pallas import tpu_sc as plsc

NUM_CLASSES = 1000000
EMB_CHANNEL = 64
BATCH = 16384

NC = 2
NS = 16
NW = NC * NS
B_PER_W = BATCH // NW


@functools.lru_cache(maxsize=1)
def _build_gather():
    mesh = plsc.VectorSubcoreMesh(core_axis_name="c", subcore_axis_name="s")

    @functools.partial(
        pl.kernel,
        mesh=mesh,
        out_type=jax.ShapeDtypeStruct((BATCH, EMB_CHANNEL), jnp.float32),
        scratch_types=[
            pltpu.VMEM_SHARED((NS, B_PER_W), jnp.int32),
            pltpu.SMEM((B_PER_W,), jnp.int32),
            pltpu.VMEM((B_PER_W, EMB_CHANNEL), jnp.float32),
            pltpu.SemaphoreType.DMA,
        ],
    )
    def emb_gather(table_hbm, idx_hbm, out_hbm, idx_sh, idx_s, rows_v, sem):
        cid = lax.axis_index("c")
        sid = lax.axis_index("s")
        wid = sid * NC + cid
        base = wid * B_PER_W
        pltpu.sync_copy(idx_hbm.at[pl.ds(base, B_PER_W)], idx_sh.at[sid])
        pltpu.sync_copy(idx_sh.at[sid], idx_s)

        def body(j, _):
            i = idx_s[j]
            pltpu.async_copy(table_hbm.at[i >> 3, i & 7], rows_v.at[j], sem)
            return _

        lax.fori_loop(0, B_PER_W, body, None)
        pltpu.make_async_copy(
            out_hbm.at[pl.ds(base, B_PER_W)], rows_v, sem
        ).wait()
        pltpu.sync_copy(rows_v, out_hbm.at[pl.ds(base, B_PER_W)])

    return emb_gather


def kernel(class_labels, label_emb_weight):
    idx = class_labels.astype(jnp.int32)
    table3 = label_emb_weight.reshape(NUM_CLASSES // 8, 8, EMB_CHANNEL)
    return _build_gather()(table3, idx)
